# baseline (speedup 1.0000x reference)
import functools

import jax
import jax.numpy as jnp
from jax import lax
from jax.experimental import pallas as pl
from jax.experimental.pallas import tpu as pltpu
from jax.experimental.pallas import tpu_sc as plsc

_V = 1000000
_D = 64
_B = 16384

_NC = 2
_NS = 16
_NW = _NC * _NS
_BPW = _B // _NW
_CHUNK = 128
_NCH = _BPW // _CHUNK


def _sc_partials(cw, xw, cemb, xemb):
    mesh = plsc.VectorSubcoreMesh(core_axis_name="c", subcore_axis_name="s")

    @functools.partial(
        pl.kernel,
        mesh=mesh,
        compiler_params=pltpu.CompilerParams(
            needs_layout_passes=False, use_tc_tiling_on_sc=False),
        out_type=jax.ShapeDtypeStruct((2 * _NW, 16), jnp.float32),
        scratch_types=[
            pltpu.VMEM((_NCH, _CHUNK), jnp.int32),
            pltpu.VMEM((_NCH, _CHUNK), jnp.int32),
            pltpu.VMEM((_BPW, _D), jnp.float32),
            pltpu.VMEM((_BPW, _D), jnp.float32),
            pltpu.VMEM((16,), jnp.float32),
            pltpu.VMEM((16,), jnp.float32),
            pltpu.SemaphoreType.DMA,
        ],
    )
    def k(cw_hbm, xw_hbm, cemb_hbm, xemb_hbm, out_hbm,
          idxc, idxx, rowsc, rowsx, resm, rese, sem):
        wid = lax.axis_index("s") * _NC + lax.axis_index("c")
        base = wid * _BPW

        for j in range(_NCH):
            pltpu.sync_copy(cw_hbm.at[pl.ds(base + j * _CHUNK, _CHUNK)],
                            idxc.at[j])
            pltpu.sync_copy(xw_hbm.at[pl.ds(base + j * _CHUNK, _CHUNK)],
                            idxx.at[j])

        cps = []
        for j in range(_NCH):
            cps.append(pltpu.async_copy(
                cemb_hbm.at[idxc.at[j]],
                rowsc.at[pl.ds(j * _CHUNK, _CHUNK)], sem))
            cps.append(pltpu.async_copy(
                xemb_hbm.at[idxx.at[j]],
                rowsx.at[pl.ds(j * _CHUNK, _CHUNK)], sem))
        for cp in cps:
            cp.wait()

        lanes = lax.iota(jnp.int32, 16)
        zero = jnp.zeros((16,), jnp.float32)

        def tile_body(t, carry):
            sm, se = carry
            base = t * 16
            dvec = zero
            for i in range(16):
                r = base + i
                p = zero
                for kk in range(_D // 16):
                    a = rowsc[r, pl.ds(kk * 16, 16)]
                    b = rowsx[r, pl.ds(kk * 16, 16)]
                    p = p + a * b
                dot = jnp.sum(p)
                dvec = dvec + jnp.where(lanes == i, dot, 0.0)
            return sm + dvec, se + jnp.exp(dvec)

        sm, se = lax.fori_loop(0, _BPW // 16, tile_body, (zero, zero))
        resm[...] = sm
        rese[...] = se
        pltpu.sync_copy(resm, out_hbm.at[wid])
        pltpu.sync_copy(rese, out_hbm.at[_NW + wid])

    return k(cw, xw, cemb, xemb)


def _tc_finish(p_ref, o_ref):
    x = p_ref[...]
    t = jnp.sum(x[:_NW])
    s = jnp.sum(x[_NW:])
    o_ref[...] = jnp.reshape(jnp.float32(_B) * jnp.log(s) - t, (1, 1))


def kernel(center_word, context_word, center_emb, context_emb):
    cw = center_word.astype(jnp.int32)
    xw = context_word.astype(jnp.int32)
    parts = _sc_partials(cw, xw, center_emb, context_emb)
    loss = pl.pallas_call(
        _tc_finish,
        out_shape=jax.ShapeDtypeStruct((1, 1), jnp.float32),
    )(parts)
    return loss[0, 0]


# speedup vs baseline: 1.0000x; 1.0000x over previous
import functools

import jax
import jax.numpy as jnp
from jax import lax
from jax.experimental import pallas as pl
from jax.experimental.pallas import tpu as pltpu
from jax.experimental.pallas import tpu_sc as plsc

_V = 1000000
_D = 64
_B = 16384

_NC = 2
_NS = 16
_NW = _NC * _NS
_BPW = _B // _NW
_HB = _BPW // 2


def _sc_partials(cw, xw, a2, b2):
    mesh = plsc.VectorSubcoreMesh(core_axis_name="c", subcore_axis_name="s")

    @functools.partial(
        pl.kernel,
        mesh=mesh,
        compiler_params=pltpu.CompilerParams(
            needs_layout_passes=False, use_tc_tiling_on_sc=True),
        out_type=jax.ShapeDtypeStruct((2 * _NW, 16), jnp.float32),
        scratch_types=[
            pltpu.VMEM((_BPW,), jnp.int32),
            pltpu.VMEM((_BPW,), jnp.int32),
            pltpu.VMEM((_BPW,), jnp.int32),
            pltpu.VMEM((_BPW,), jnp.int32),
            pltpu.VMEM((_HB, 128), jnp.float32),
            pltpu.VMEM((_HB, 128), jnp.float32),
            pltpu.VMEM((16,), jnp.float32),
            pltpu.VMEM((16,), jnp.float32),
            pltpu.SemaphoreType.DMA,
        ],
    )
    def k(cw_hbm, xw_hbm, a2_hbm, b2_hbm, out_hbm,
          idxc, idxx, prc, prx, ga, gb, resm, rese, sem):
        wid = lax.axis_index("s") * _NC + lax.axis_index("c")
        base = wid * _BPW

        pltpu.sync_copy(cw_hbm.at[pl.ds(base, _BPW)], idxc)
        pltpu.sync_copy(xw_hbm.at[pl.ds(base, _BPW)], idxx)
        for t in range(_BPW // 16):
            s = pl.ds(t * 16, 16)
            prc[s] = idxc[s] >> 1
            prx[s] = idxx[s] >> 1

        lanes = lax.iota(jnp.int32, 16)
        zero = jnp.zeros((16,), jnp.float32)
        one = jnp.int32(1)

        def half(h, carry):
            sm, se = carry
            cpa = pltpu.async_copy(a2_hbm.at[prc.at[pl.ds(h * _HB, _HB)]],
                                   ga, sem)
            cpb = pltpu.async_copy(b2_hbm.at[prx.at[pl.ds(h * _HB, _HB)]],
                                   gb, sem)
            cpa.wait()
            cpb.wait()

            def grp_body(g, carry2):
                sm2, se2 = carry2
                pa = idxc[pl.ds(h * _HB + g * 16, 16)] & one
                pb = idxx[pl.ds(h * _HB + g * 16, 16)] & one
                dvec = zero
                for j in range(16):
                    r = g * 16 + j
                    lane_j = lanes == j
                    ha = jnp.sum(jnp.where(lane_j, pa, 0)) == one
                    hb = jnp.sum(jnp.where(lane_j, pb, 0)) == one
                    p = zero
                    for kk in range(4):
                        alo = ga[r, pl.ds(kk * 16, 16)]
                        ahi = ga[r, pl.ds(64 + kk * 16, 16)]
                        blo = gb[r, pl.ds(kk * 16, 16)]
                        bhi = gb[r, pl.ds(64 + kk * 16, 16)]
                        p = p + (jnp.where(ha, ahi, alo)
                                 * jnp.where(hb, bhi, blo))
                    dot = jnp.sum(p)
                    dvec = dvec + jnp.where(lane_j, dot, 0.0)
                return sm2 + dvec, se2 + jnp.exp(dvec)

            return lax.fori_loop(0, _HB // 16, grp_body, (sm, se))

        sm, se = lax.fori_loop(0, 2, half, (zero, zero))
        resm[...] = sm
        rese[...] = se
        pltpu.sync_copy(resm, out_hbm.at[wid])
        pltpu.sync_copy(rese, out_hbm.at[_NW + wid])

    return k(cw, xw, a2, b2)


def _tc_finish(p_ref, o_ref):
    x = p_ref[...]
    t = jnp.sum(x[:_NW])
    s = jnp.sum(x[_NW:])
    o_ref[...] = jnp.reshape(jnp.float32(_B) * jnp.log(s) - t, (1, 1))


def kernel(center_word, context_word, center_emb, context_emb):
    cw = center_word.astype(jnp.int32)
    xw = context_word.astype(jnp.int32)
    parts = _sc_partials(cw, xw,
                         center_emb.reshape(_V // 2, 128),
                         context_emb.reshape(_V // 2, 128))
    loss = pl.pallas_call(
        _tc_finish,
        out_shape=jax.ShapeDtypeStruct((1, 1), jnp.float32),
    )(parts)
    return loss[0, 0]


# speedup vs baseline: 2.4058x; 2.4057x over previous
import functools

import jax
import jax.numpy as jnp
from jax import lax
from jax.experimental import pallas as pl
from jax.experimental.pallas import tpu as pltpu
from jax.experimental.pallas import tpu_sc as plsc

_V = 1000000
_D = 64
_B = 16384

_NC = 2
_NS = 16
_NW = _NC * _NS
_BPW = _B // _NW
_HB = _BPW // 2


def _sc_partials(cw, xw, a2, b2):
    mesh = plsc.VectorSubcoreMesh(core_axis_name="c", subcore_axis_name="s")

    @functools.partial(
        pl.kernel,
        mesh=mesh,
        compiler_params=pltpu.CompilerParams(
            needs_layout_passes=False, use_tc_tiling_on_sc=True),
        out_type=jax.ShapeDtypeStruct((2 * _NW, 16), jnp.float32),
        scratch_types=[
            pltpu.VMEM((_BPW,), jnp.int32),
            pltpu.VMEM((_BPW,), jnp.int32),
            pltpu.VMEM((32, 1, _D), jnp.float32),
            pltpu.VMEM((32, 1, _D), jnp.float32),
            pltpu.VMEM((16,), jnp.float32),
            pltpu.VMEM((16,), jnp.float32),
            pltpu.SemaphoreType.DMA,
        ],
    )
    def k(cw_hbm, xw_hbm, a2_hbm, b2_hbm, out_hbm,
          idxc, idxx, ga, gb, resm, rese, sem):
        wid = lax.axis_index("s") * _NC + lax.axis_index("c")
        base = wid * _BPW

        pltpu.sync_copy(cw_hbm.at[pl.ds(base, _BPW)], idxc)
        pltpu.sync_copy(xw_hbm.at[pl.ds(base, _BPW)], idxx)

        lanes = lax.iota(jnp.int32, 16)
        zero = jnp.zeros((16,), jnp.float32)

        def chunk(h, carry):
            sm, se = carry
            cpa = pltpu.async_copy(a2_hbm.at[idxc.at[pl.ds(h * 32, 32)]],
                                   ga, sem)
            cpb = pltpu.async_copy(b2_hbm.at[idxx.at[pl.ds(h * 32, 32)]],
                                   gb, sem)
            cpa.wait()
            cpb.wait()

            def grp_body(g, carry2):
                sm2, se2 = carry2
                dvec = zero
                for j in range(16):
                    r = g * 16 + j
                    lane_j = lanes == j
                    p = zero
                    for kk in range(4):
                        a = ga[r, 0, pl.ds(kk * 16, 16)]
                        b = gb[r, 0, pl.ds(kk * 16, 16)]
                        p = p + a * b
                    dot = jnp.sum(p)
                    dvec = dvec + jnp.where(lane_j, dot, 0.0)
                return sm2 + dvec, se2 + jnp.exp(dvec)

            return lax.fori_loop(0, 2, grp_body, (sm, se))

        sm, se = lax.fori_loop(0, _BPW // 32, chunk, (zero, zero))
        resm[...] = sm
        rese[...] = se
        pltpu.sync_copy(resm, out_hbm.at[wid])
        pltpu.sync_copy(rese, out_hbm.at[_NW + wid])

    return k(cw, xw, a2, b2)


def _tc_finish(p_ref, o_ref):
    x = p_ref[...]
    t = jnp.sum(x[:_NW])
    s = jnp.sum(x[_NW:])
    o_ref[...] = jnp.reshape(jnp.float32(_B) * jnp.log(s) - t, (1, 1))


def kernel(center_word, context_word, center_emb, context_emb):
    cw = center_word.astype(jnp.int32)
    xw = context_word.astype(jnp.int32)
    parts = _sc_partials(cw, xw,
                         center_emb.reshape(_V, 1, _D),
                         context_emb.reshape(_V, 1, _D))
    loss = pl.pallas_call(
        _tc_finish,
        out_shape=jax.ShapeDtypeStruct((1, 1), jnp.float32),
    )(parts)
    return loss[0, 0]
